# baseline (speedup 1.0000x reference)
import functools

import jax
import jax.numpy as jnp
from jax import lax
from jax.experimental import pallas as pl
from jax.experimental.pallas import tpu as pltpu
from jax.experimental.pallas import tpu_sc as plsc

_C = 128
_GPB = 2
_NBUF = 3
_DIST = 2


@functools.lru_cache(maxsize=None)
def _build_sc_gather(n, v, d, dtype_name):
    dtype = jnp.dtype(dtype_name)
    info = plsc.get_sparse_core_info()
    nc, ns = info.num_cores, info.num_subcores
    nw = nc * ns
    rows_buf = _GPB * _C
    full = n // _C
    tail = n % _C
    base = full // nw
    extra = full % nw
    nbig = base // _GPB
    assert base % _GPB == 0 and nbig % _NBUF == 0 and nbig > _DIST
    assert tail % 8 == 0 and extra < nw - 1

    len_lo = base * _C
    len_hi = (base + 1) * _C
    len_last = base * _C + tail

    mesh = plsc.VectorSubcoreMesh(core_axis_name="c", subcore_axis_name="s")

    scratch = [
        pltpu.VMEM((len_hi,), jnp.int32),
    ] + [
        pltpu.VMEM((rows_buf, d), dtype)
        for _ in range(_NBUF)
    ] + [
        pltpu.VMEM((max(tail, 8), d), dtype),
        pltpu.SemaphoreType.DMA((_NBUF,)),
        pltpu.SemaphoreType.DMA((_NBUF,)),
        pltpu.SemaphoreType.DMA((_NBUF,)),
        pltpu.VMEM_SHARED((v, d), dtype),
    ]

    out_t = jax.ShapeDtypeStruct((n, d), dtype)

    @functools.partial(
        pl.kernel,
        mesh=mesh,
        out_type=(out_t, out_t),
        scratch_types=scratch,
    )
    def gather_kernel(idx_hbm, table_hbm, out0_hbm, out1_hbm, idx_all,
                      *rest):
        rows = rest[:_NBUF]
        rows_t, gsem, s0sem, s1sem, tab_sp = rest[_NBUF:]
        outs = (out0_hbm, out1_hbm)
        ssems = (s0sem, s1sem)
        sid = lax.axis_index("s")
        w = sid * nc + lax.axis_index("c")
        s = base * w + jnp.minimum(w, extra)
        idx_start = s * _C

        @pl.when(sid == 0)
        def _():
            pltpu.sync_copy(table_hbm, tab_sp)

        @pl.when(w < extra)
        def _():
            pltpu.sync_copy(idx_hbm.at[pl.ds(idx_start, len_hi)],
                            idx_all.at[pl.ds(0, len_hi)])

        @pl.when(jnp.logical_and(w >= extra, w < nw - 1))
        def _():
            pltpu.sync_copy(idx_hbm.at[pl.ds(idx_start, len_lo)],
                            idx_all.at[pl.ds(0, len_lo)])

        @pl.when(w == nw - 1)
        def _():
            pltpu.sync_copy(idx_hbm.at[pl.ds(idx_start, len_last)],
                            idx_all.at[pl.ds(0, len_last)])

        plsc.subcore_barrier()

        def gather_big(j, b):
            for h in range(_GPB):
                pltpu.async_copy(
                    tab_sp.at[idx_all.at[pl.ds((j * _GPB + h) * _C, _C)]],
                    rows[b].at[pl.ds(h * _C, _C), :], gsem.at[b])

        def wait_gather_big(j, b):
            for h in range(_GPB):
                pltpu.make_async_copy(
                    tab_sp.at[idx_all.at[pl.ds((j * _GPB + h) * _C, _C)]],
                    rows[b].at[pl.ds(h * _C, _C), :], gsem.at[b]).wait()

        def wait_store_big(b):
            for o in range(2):
                pltpu.make_async_copy(rows[b],
                                      outs[o].at[pl.ds(0, rows_buf), :],
                                      ssems[o].at[b]).wait()

        for j in range(_DIST):
            gather_big(j, j % _NBUF)

        def group(g, carry):
            for bs in range(_NBUF):
                j = g * _NBUF + bs
                b = bs
                wait_gather_big(j, b)
                row0 = s * _C + j * rows_buf
                for o in range(2):
                    pltpu.async_copy(rows[b],
                                     outs[o].at[pl.ds(row0, rows_buf), :],
                                     ssems[o].at[b])
                j2 = j + _DIST
                b2 = (bs + _DIST) % _NBUF

                @pl.when(j2 < nbig)
                def _(j2=j2, b2=b2):
                    @pl.when(j2 >= _NBUF)
                    def _():
                        wait_store_big(b2)
                    gather_big(j2, b2)
            return carry

        lax.fori_loop(0, nbig // _NBUF, group, 0)

        for b in range(_NBUF):
            wait_store_big(b)

        @pl.when(w < extra)
        def _():
            pltpu.async_copy(
                tab_sp.at[idx_all.at[pl.ds(base * _C, _C)]],
                rows[0].at[pl.ds(0, _C), :], gsem.at[0]).wait()
            for o in range(2):
                pltpu.sync_copy(rows[0].at[pl.ds(0, _C), :],
                                outs[o].at[pl.ds((s + base) * _C, _C), :])

        if tail:
            @pl.when(w == nw - 1)
            def _():
                pltpu.async_copy(
                    tab_sp.at[idx_all.at[pl.ds(base * _C, tail)]],
                    rows_t.at[pl.ds(0, tail), :], gsem.at[0]).wait()
                for o in range(2):
                    pltpu.sync_copy(rows_t.at[pl.ds(0, tail), :],
                                    outs[o].at[pl.ds(full * _C, tail), :])

    return gather_kernel


def kernel(atom_types, pos, table):
    idx = jnp.reshape(atom_types, (-1,))
    tab = table.astype(pos.dtype)
    n = idx.shape[0]
    v, d = tab.shape
    out0, out1 = _build_sc_gather(n, v, d, str(tab.dtype))(idx, tab)
    return (out0, out1)


# speedup vs baseline: 1.0534x; 1.0534x over previous
import functools

import jax
import jax.numpy as jnp
from jax import lax
from jax.experimental import pallas as pl
from jax.experimental.pallas import tpu as pltpu
from jax.experimental.pallas import tpu_sc as plsc

_C = 128
_NBUF = 6
_DIST = 3


@functools.lru_cache(maxsize=None)
def _build_sc_gather(n, v, d, dtype_name):
    dtype = jnp.dtype(dtype_name)
    info = plsc.get_sparse_core_info()
    nc, ns = info.num_cores, info.num_subcores
    nw = nc * ns
    full = n // _C
    tail = n % _C
    base = full // nw
    extra = full % nw
    assert base >= _NBUF and tail % 8 == 0 and extra < nw - 1
    assert base % _NBUF == 0

    len_lo = base * _C
    len_hi = (base + 1) * _C
    len_last = base * _C + tail

    mesh = plsc.VectorSubcoreMesh(core_axis_name="c", subcore_axis_name="s")

    scratch = [
        pltpu.VMEM((len_hi,), jnp.int32),
    ] + [
        pltpu.VMEM((_C, d), dtype)
        for _ in range(_NBUF)
    ] + [
        pltpu.VMEM((max(tail, 8), d), dtype),
        pltpu.SemaphoreType.DMA((_NBUF,)),
        pltpu.SemaphoreType.DMA((_NBUF,)),
        pltpu.SemaphoreType.DMA((_NBUF,)),
        pltpu.VMEM_SHARED((v, d), dtype),
    ]

    out_t = jax.ShapeDtypeStruct((n, d), dtype)

    @functools.partial(
        pl.kernel,
        mesh=mesh,
        out_type=(out_t, out_t),
        scratch_types=scratch,
    )
    def gather_kernel(idx_hbm, table_hbm, out0_hbm, out1_hbm, idx_all,
                      *rest):
        rows = rest[:_NBUF]
        rows_t, gsem, s0sem, s1sem, tab_sp = rest[_NBUF:]
        outs = (out0_hbm, out1_hbm)
        ssems = (s0sem, s1sem)
        sid = lax.axis_index("s")
        w = sid * nc + lax.axis_index("c")
        s = base * w + jnp.minimum(w, extra)
        idx_start = s * _C

        @pl.when(sid == 0)
        def _():
            pltpu.sync_copy(table_hbm, tab_sp)

        @pl.when(w < extra)
        def _():
            pltpu.sync_copy(idx_hbm.at[pl.ds(idx_start, len_hi)],
                            idx_all.at[pl.ds(0, len_hi)])

        @pl.when(jnp.logical_and(w >= extra, w < nw - 1))
        def _():
            pltpu.sync_copy(idx_hbm.at[pl.ds(idx_start, len_lo)],
                            idx_all.at[pl.ds(0, len_lo)])

        @pl.when(w == nw - 1)
        def _():
            pltpu.sync_copy(idx_hbm.at[pl.ds(idx_start, len_last)],
                            idx_all.at[pl.ds(0, len_last)])

        plsc.subcore_barrier()

        def gather_async(c, b):
            return pltpu.async_copy(
                tab_sp.at[idx_all.at[pl.ds(c * _C, _C)]], rows[b],
                gsem.at[b])

        def wait_gather(c, b):
            pltpu.make_async_copy(
                tab_sp.at[idx_all.at[pl.ds(c * _C, _C)]], rows[b],
                gsem.at[b]).wait()

        def wait_store(b):
            for o in range(2):
                pltpu.make_async_copy(rows[b], outs[o].at[pl.ds(0, _C), :],
                                      ssems[o].at[b]).wait()

        for c in range(_DIST):
            gather_async(c, c % _NBUF)

        nchunks = base + jnp.where(w < extra, 1, 0)

        def group(j, carry):
            for b in range(_NBUF):
                c = j * _NBUF + b
                wait_gather(c, b)
                for o in range(2):
                    pltpu.async_copy(rows[b],
                                     outs[o].at[pl.ds((s + c) * _C, _C), :],
                                     ssems[o].at[b])
                c2 = c + _DIST
                b2 = (b + _DIST) % _NBUF

                @pl.when(c2 < nchunks)
                def _(c2=c2, b2=b2):
                    @pl.when(c2 >= _NBUF)
                    def _():
                        wait_store(b2)
                    gather_async(c2, b2)
            return carry

        lax.fori_loop(0, base // _NBUF, group, 0)

        @pl.when(w < extra)
        def _():
            b = base % _NBUF
            wait_gather(base, b)
            for o in range(2):
                pltpu.sync_copy(rows[b],
                                outs[o].at[pl.ds((s + base) * _C, _C), :])
            for bb in range(_NBUF):
                if bb != base % _NBUF:
                    wait_store(bb)

        @pl.when(w >= extra)
        def _():
            for bb in range(_NBUF):
                wait_store(bb)

        if tail:
            @pl.when(w == nw - 1)
            def _():
                pltpu.async_copy(
                    tab_sp.at[idx_all.at[pl.ds(base * _C, tail)]],
                    rows_t.at[pl.ds(0, tail), :], gsem.at[0]).wait()
                for o in range(2):
                    pltpu.sync_copy(rows_t.at[pl.ds(0, tail), :],
                                    outs[o].at[pl.ds(full * _C, tail), :])

    return gather_kernel


def kernel(atom_types, pos, table):
    idx = jnp.reshape(atom_types, (-1,))
    tab = table.astype(pos.dtype)
    n = idx.shape[0]
    v, d = tab.shape
    out0, out1 = _build_sc_gather(n, v, d, str(tab.dtype))(idx, tab)
    return (out0, out1)
